# Initial kernel scaffold; baseline (speedup 1.0000x reference)
import functools

import jax
import jax.numpy as jnp
from jax import lax
from jax.experimental import pallas as pl
from jax.experimental.pallas import tpu as pltpu
from jax.experimental.pallas import tpu_sc as plsc

N = 320000
D = 128
NSEG = 512
NCLS = 10

NC = 2
NS = 16
NW = NC * NS
RW = N // NW
RB = 250
NB = RW // RB
NV = D // 16

_NEG_INF = jnp.float32(-jnp.inf)


def _sc_body(x_hbm, ids_hbm, watt_hbm, part_hbm, ids_v, w_v, xblk_v, acc_v):
    cid = lax.axis_index("c")
    sid = lax.axis_index("s")
    wid = sid * NC + cid
    w0 = wid * RW

    pltpu.sync_copy(ids_hbm.at[pl.ds(w0, RW)], ids_v)
    pltpu.sync_copy(watt_hbm, w_v)

    neg = jnp.full((16,), _NEG_INF, jnp.float32)

    def init_body(s, carry):
        for v in range(NV):
            acc_v[s, pl.ds(v * 16, 16)] = neg
        return carry

    lax.fori_loop(0, NSEG, init_body, 0)

    wvec = [w_v[pl.ds(v * 16, 16)] for v in range(NV)]

    def block_body(b, carry):
        blk0 = w0 + b * RB
        pltpu.sync_copy(x_hbm.at[pl.ds(blk0, RB)], xblk_v)
        lb = b * RB

        def row_body(r, c2):
            xr = [xblk_v[r, pl.ds(v * 16, 16)] for v in range(NV)]
            p = xr[0] * wvec[0]
            for v in range(1, NV):
                p = p + xr[v] * wvec[v]
            sdot = jnp.sum(p)
            sv = jnp.full((16,), sdot, jnp.float32)
            gate = (1.0 / (1.0 + jnp.exp(-sv)) + 1.0) * jnp.float32(0.5)
            seg = ids_v[lb + r]
            for v in range(NV):
                cur = acc_v[seg, pl.ds(v * 16, 16)]
                acc_v[seg, pl.ds(v * 16, 16)] = jnp.maximum(cur, xr[v] * gate)
            return c2

        lax.fori_loop(0, RB, row_body, 0)
        return carry

    lax.fori_loop(0, NB, block_body, 0)

    pltpu.sync_copy(acc_v, part_hbm.at[wid])


@jax.jit
def _sc_segment_pool(x, ids, watt):
    mesh = plsc.VectorSubcoreMesh(core_axis_name="c", subcore_axis_name="s")
    fn = pl.kernel(
        _sc_body,
        out_type=jax.ShapeDtypeStruct((NW, NSEG, D), jnp.float32),
        mesh=mesh,
        scratch_types=[
            pltpu.VMEM((RW,), jnp.int32),
            pltpu.VMEM((D,), jnp.float32),
            pltpu.VMEM((RB, D), jnp.float32),
            pltpu.VMEM((NSEG, D), jnp.float32),
        ],
    )
    return fn(x, ids, watt)


def _combine_body(p_ref, w_ref, o_ref):
    hg = jnp.max(p_ref[...], axis=0)
    o_ref[...] = jax.lax.dot_general(
        hg, w_ref[...], (((1,), (1,)), ((), ())),
        preferred_element_type=jnp.float32)


@jax.jit
def _combine(part, w_read):
    return pl.pallas_call(
        _combine_body,
        out_shape=jax.ShapeDtypeStruct((NSEG, NCLS), jnp.float32),
    )(part, w_read)


def kernel(x, batch, W_att, W_read):
    ids = batch.astype(jnp.int32)
    watt = W_att.reshape(D)
    part = _sc_segment_pool(x, ids, watt)
    return _combine(part, W_read)


# speedup vs baseline: 2.2429x; 2.2429x over previous
import functools

import jax
import jax.numpy as jnp
from jax import lax
from jax.experimental import pallas as pl
from jax.experimental.pallas import tpu as pltpu
from jax.experimental.pallas import tpu_sc as plsc

N = 320000
D = 128
NSEG = 512
NCLS = 10

NC = 2
NS = 16
NW = NC * NS
RW = N // NW
RB = 200
NB = RW // RB
NV = D // 16

_NEG_INF = float("-inf")


def _sc_body(x_hbm, ids_hbm, watt_hbm, part_hbm, ids_v, w_v, xblk_v, acc_v):
    cid = lax.axis_index("c")
    sid = lax.axis_index("s")
    wid = sid * NC + cid
    w0 = wid * RW

    pltpu.sync_copy(ids_hbm.at[pl.ds(w0, RW)], ids_v.at[pl.ds(0, RW)])
    pltpu.sync_copy(watt_hbm, w_v)

    neg = jnp.full((16,), _NEG_INF, jnp.float32)

    def init_body(s, carry):
        for v in range(NV):
            acc_v[s, pl.ds(v * 16, 16)] = neg
        return carry

    lax.fori_loop(0, NSEG, init_body, 0)

    wvec = [w_v[pl.ds(v * 16, 16)] for v in range(NV)]

    lanes = jnp.arange(16, dtype=jnp.int32)
    perms = [(lanes ^ (1 << k))[:, None] for k in range(4)]
    gdn = lax.GatherDimensionNumbers(
        offset_dims=(), collapsed_slice_dims=(0,), start_index_map=(0,))

    def _shuffle(v, pm):
        return lax.gather(v, pm, gdn, slice_sizes=(1,),
                          mode=lax.GatherScatterMode.PROMISE_IN_BOUNDS)

    def block_body(b, carry):
        blk0 = w0 + b * RB
        pltpu.sync_copy(x_hbm.at[pl.ds(blk0, RB)], xblk_v)
        lb = b * RB

        def row_body(r, c2):
            xr = [xblk_v[r, pl.ds(v * 16, 16)] for v in range(NV)]
            p = xr[0] * wvec[0]
            for v in range(1, NV):
                p = p + xr[v] * wvec[v]
            for pm in perms:
                p = p + _shuffle(p, pm)
            gate = (1.0 / (1.0 + jnp.exp(-p)) + 1.0) * jnp.float32(0.5)
            seg = ids_v[pl.ds(lb + r, 16)][0]
            for v in range(NV):
                cur = acc_v[seg, pl.ds(v * 16, 16)]
                acc_v[seg, pl.ds(v * 16, 16)] = jnp.maximum(cur, xr[v] * gate)
            return c2

        lax.fori_loop(0, RB, row_body, 0)
        return carry

    lax.fori_loop(0, NB, block_body, 0)

    pltpu.sync_copy(acc_v, part_hbm.at[wid])


@jax.jit
def _sc_segment_pool(x, ids, watt):
    mesh = plsc.VectorSubcoreMesh(core_axis_name="c", subcore_axis_name="s")
    fn = pl.kernel(
        _sc_body,
        out_type=jax.ShapeDtypeStruct((NW, NSEG, D), jnp.float32),
        mesh=mesh,
        scratch_types=[
            pltpu.VMEM((RW + 16,), jnp.int32),
            pltpu.VMEM((D,), jnp.float32),
            pltpu.VMEM((RB, D), jnp.float32),
            pltpu.VMEM((NSEG, D), jnp.float32),
        ],
    )
    return fn(x, ids, watt)


def _combine_body(p_ref, w_ref, o_ref):
    hg = jnp.max(p_ref[...], axis=0)
    o_ref[...] = jax.lax.dot_general(
        hg, w_ref[...], (((1,), (1,)), ((), ())),
        preferred_element_type=jnp.float32)


@jax.jit
def _combine(part, w_read):
    return pl.pallas_call(
        _combine_body,
        out_shape=jax.ShapeDtypeStruct((NSEG, NCLS), jnp.float32),
    )(part, w_read)


def kernel(x, batch, W_att, W_read):
    ids = batch.astype(jnp.int32)
    watt = W_att.reshape(D)
    part = _sc_segment_pool(x, ids, watt)
    return _combine(part, W_read)
